# baseline (speedup 1.0000x reference)
import jax
import jax.numpy as jnp
from jax import lax
from jax.experimental import pallas as pl
from jax.experimental.pallas import tpu as pltpu
from jax.experimental.pallas import tpu_sc as plsc

NC = 2
NS = 16
NW = NC * NS
LANES = 16
CHUNK = 128


def _make_body(BPW, NCH, D):
    nvec = D // LANES

    def body(x_hbm, idx_hbm, center_hbm, out_hbm,
             idx_v, x_v, rows_v, acc_v, sem0, sem1):
        wid = lax.axis_index("s") * NC + lax.axis_index("c")
        base = wid * BPW
        sems = [sem0, sem1]

        pltpu.sync_copy(idx_hbm.at[pl.ds(wid * NCH, NCH)], idx_v)

        def fire(c):
            return pltpu.async_copy(
                center_hbm.at[idx_v.at[c]], rows_v.at[c % 2], sems[c % 2])

        copies = {0: fire(0), 1: fire(1)}
        pltpu.sync_copy(x_hbm.at[pl.ds(base, BPW)], x_v)

        zero = jnp.zeros((LANES,), jnp.float32)
        accs = (zero,) * nvec

        for c in range(NCH):
            copies[c].wait()
            buf = c % 2

            def item_body(i, a, _c=c, _buf=buf):
                new = []
                for j in range(nvec):
                    xv = x_v[_c * CHUNK + i, pl.ds(j * LANES, LANES)]
                    cv = rows_v[_buf, i, pl.ds(j * LANES, LANES)]
                    d = xv - cv
                    new.append(a[j] + d * d)
                return tuple(new)

            accs = lax.fori_loop(0, CHUNK, item_body, accs)
            if c + 2 < NCH:
                copies[c + 2] = fire(c + 2)

        total = accs[0]
        for j in range(1, nvec):
            total = total + accs[j]
        acc_v[...] = total
        pltpu.sync_copy(acc_v, out_hbm.at[wid])

    return body


def _pad_transpose_block(in_ref, out_ref):
    out_ref[...] = in_ref[...].T


def _relayout_rows(a_t, n_rows, bk):
    grid = (n_rows + bk - 1) // bk
    return pl.pallas_call(
        _pad_transpose_block,
        grid=(grid,),
        in_specs=[pl.BlockSpec((64, bk), lambda g: (0, g))],
        out_specs=pl.BlockSpec((bk, 64), lambda g: (g, 0)),
        out_shape=jax.ShapeDtypeStruct((2 * n_rows, 64), jnp.float32),
    )(a_t)


@jax.jit
def kernel(inputs, labels, center):
    B, D = inputs.shape
    BPW = B // NW
    NCH = BPW // CHUNK

    idx2d = labels.astype(jnp.int32).reshape(NW * NCH, CHUNK)
    x_p = _relayout_rows(inputs.T, B, 8192)
    center_p = _relayout_rows(center.T, center.shape[0], 8192)

    mesh = plsc.VectorSubcoreMesh(core_axis_name="c", subcore_axis_name="s")
    body = _make_body(BPW, NCH, D)

    partials = pl.kernel(
        body,
        out_type=jax.ShapeDtypeStruct((NW, LANES), jnp.float32),
        mesh=mesh,
        scratch_types=[
            pltpu.VMEM((NCH, CHUNK), jnp.int32),
            pltpu.VMEM((BPW, 64), jnp.float32),
            pltpu.VMEM((2, CHUNK, 64), jnp.float32),
            pltpu.VMEM((LANES,), jnp.float32),
            pltpu.SemaphoreType.DMA,
            pltpu.SemaphoreType.DMA,
        ],
        compiler_params=pltpu.CompilerParams(use_tc_tiling_on_sc=False),
    )(x_p, idx2d, center_p)

    return jnp.sum(partials) * (1.0 / (B * D))


# speedup vs baseline: 1.5749x; 1.5749x over previous
import jax
import jax.numpy as jnp
from jax import lax
from jax.experimental import pallas as pl
from jax.experimental.pallas import tpu as pltpu
from jax.experimental.pallas import tpu_sc as plsc

NC = 2
NS = 16
NW = NC * NS
LANES = 16
CHUNK = 128
BKH = 1024


def _pack_rows_block(a_ref, b_ref, out_ref):
    out_ref[...] = jnp.concatenate([a_ref[...].T, b_ref[...].T], axis=1)


def _pack_rows(a_t, k):
    grid = k // BKH
    return pl.pallas_call(
        _pack_rows_block,
        grid=(grid,),
        in_specs=[
            pl.BlockSpec((64, BKH), lambda g: (0, g)),
            pl.BlockSpec((64, BKH), lambda g: (0, k // BKH + g)),
        ],
        out_specs=pl.BlockSpec((BKH, 128), lambda g: (g, 0)),
        out_shape=jax.ShapeDtypeStruct((k, 128), jnp.float32),
    )(a_t, a_t)


def _make_body(BPW, NCH, D):
    nvec = D // LANES

    def body(x_hbm, idx_hbm, colb_hbm, center_hbm, out_hbm,
             idx_v, x_v, rows_v, colb_v, acc_v, sem0, sem1):
        wid = lax.axis_index("s") * NC + lax.axis_index("c")
        sems = [sem0, sem1]

        pltpu.sync_copy(idx_hbm.at[pl.ds(wid * NCH, NCH)], idx_v)

        def fire(c):
            return pltpu.async_copy(
                center_hbm.at[idx_v.at[c]], rows_v.at[c % 2], sems[c % 2])

        copies = {0: fire(0), 1: fire(1)}
        xrow0 = (wid % NS) * BPW
        xcol0 = (wid // NS) * D
        pltpu.sync_copy(x_hbm.at[pl.ds(xrow0, BPW), pl.ds(xcol0, D)], x_v)
        pltpu.sync_copy(colb_hbm.at[wid], colb_v)

        zero = jnp.zeros((LANES,), jnp.float32)
        accs = (zero,) * nvec

        for c in range(NCH):
            copies[c].wait()
            buf = c % 2

            def item_body(i, a, _c=c, _buf=buf):
                r = jnp.full((LANES,), i, jnp.int32)
                cb = colb_v[_c * CHUNK + i, :]
                new = list(a)
                for s in range(nvec):
                    cv = plsc.load_gather(rows_v.at[_buf], [r, cb + s * LANES])
                    xv = x_v[_c * CHUNK + i, pl.ds(s * LANES, LANES)]
                    d = xv - cv
                    new[s] = new[s] + d * d
                return tuple(new)

            accs = lax.fori_loop(0, CHUNK, item_body, accs)
            if c + 2 < NCH:
                copies[c + 2] = fire(c + 2)

        total = accs[0]
        for j in range(1, nvec):
            total = total + accs[j]
        acc_v[...] = total
        pltpu.sync_copy(acc_v, out_hbm.at[wid])

    return body


@jax.jit
def kernel(inputs, labels, center):
    B, D = inputs.shape
    C = center.shape[0]
    BPW = B // NW
    NCH = BPW // CHUNK
    KX = B // 2
    KC = -(-(C // 2) // BKH) * BKH

    labels = labels.astype(jnp.int32)
    half = (labels >= KC).astype(jnp.int32)
    idx2d = (labels - half * KC).reshape(NW * NCH, CHUNK)
    colb = (half << 6)[:, None] + jnp.arange(LANES, dtype=jnp.int32)
    colb = colb.reshape(NW, BPW, LANES)

    x_p = _pack_rows(inputs.T, KX)
    center_p = _pack_rows(center.T, KC)

    mesh = plsc.VectorSubcoreMesh(core_axis_name="c", subcore_axis_name="s")
    body = _make_body(BPW, NCH, D)

    partials = pl.kernel(
        body,
        out_type=jax.ShapeDtypeStruct((NW, LANES), jnp.float32),
        mesh=mesh,
        scratch_types=[
            pltpu.VMEM((NCH, CHUNK), jnp.int32),
            pltpu.VMEM((BPW, 64), jnp.float32),
            pltpu.VMEM((2, CHUNK, 128), jnp.float32),
            pltpu.VMEM((BPW, LANES), jnp.int32),
            pltpu.VMEM((LANES,), jnp.float32),
            pltpu.SemaphoreType.DMA,
            pltpu.SemaphoreType.DMA,
        ],
        compiler_params=pltpu.CompilerParams(
            use_tc_tiling_on_sc=False, needs_layout_passes=False),
    )(x_p, idx2d, colb, center_p)

    return jnp.sum(partials) * (1.0 / (B * D))


# speedup vs baseline: 2.3227x; 1.4749x over previous
import jax
import jax.numpy as jnp
from jax import lax
from jax.experimental import pallas as pl
from jax.experimental.pallas import tpu as pltpu
from jax.experimental.pallas import tpu_sc as plsc

NC = 2
NS = 16
NW = NC * NS
LANES = 16
CHUNK = 128


def _pack_rows_block(a_ref, b_ref, out_ref):
    out_ref[...] = jnp.concatenate([a_ref[...].T, b_ref[...].T], axis=1)


def _pack_rows(a_t, k, bkh):
    grid = k // bkh
    return pl.pallas_call(
        _pack_rows_block,
        grid=(grid,),
        in_specs=[
            pl.BlockSpec((64, bkh), lambda g: (0, g)),
            pl.BlockSpec((64, bkh), lambda g: (0, k // bkh + g)),
        ],
        out_specs=pl.BlockSpec((bkh, 128), lambda g: (g, 0)),
        out_shape=jax.ShapeDtypeStruct((k, 128), jnp.float32),
    )(a_t, a_t)


def _make_body(BPW, NCH, D):
    nvec = D // LANES

    def body(x_hbm, idx_hbm, colb_hbm, center_hbm, out_hbm,
             idx_v, x_v, rows_v, colb_v, acc_v, sem0, sem1):
        wid = lax.axis_index("s") * NC + lax.axis_index("c")
        sems = [sem0, sem1]

        pltpu.sync_copy(idx_hbm.at[pl.ds(wid * NCH, NCH)], idx_v)

        def fire(c):
            return pltpu.async_copy(
                center_hbm.at[idx_v.at[c]], rows_v.at[c % 2], sems[c % 2])

        copies = {0: fire(0), 1: fire(1)}
        xrow0 = (wid % NS) * BPW
        xcol0 = (wid // NS) * D
        pltpu.sync_copy(x_hbm.at[pl.ds(xrow0, BPW), pl.ds(xcol0, D)], x_v)
        pltpu.sync_copy(
            colb_hbm.at[pl.ds(wid * (BPW * LANES // 128), BPW * LANES // 128)],
            colb_v)

        zero = jnp.zeros((LANES,), jnp.float32)
        iota16 = lax.iota(jnp.int32, LANES)
        accs = (zero,) * nvec

        for c in range(NCH):
            copies[c].wait()
            buf = c % 2

            def item_body(i, a, _c=c, _buf=buf):
                r = jnp.full((LANES,), i, jnp.int32)
                cb = plsc.load_gather(
                    colb_v,
                    [jnp.full((LANES,), _c * (CHUNK // 8) + (i >> 3), jnp.int32),
                     ((i & 7) << 4) + iota16])
                new = list(a)
                for s in range(nvec):
                    cv = plsc.load_gather(rows_v.at[_buf], [r, cb + s * LANES])
                    xv = x_v[_c * CHUNK + i, pl.ds(s * LANES, LANES)]
                    d = xv - cv
                    new[s] = new[s] + d * d
                return tuple(new)

            accs = lax.fori_loop(0, CHUNK, item_body, accs)
            if c + 2 < NCH:
                copies[c + 2] = fire(c + 2)

        total = accs[0]
        for j in range(1, nvec):
            total = total + accs[j]
        acc_v[...] = total
        pltpu.sync_copy(acc_v, out_hbm.at[wid])

    return body


@jax.jit
def kernel(inputs, labels, center):
    B, D = inputs.shape
    C = center.shape[0]
    BPW = B // NW
    NCH = BPW // CHUNK
    KX = B // 2
    KC = 50176

    labels = labels.astype(jnp.int32)
    half = (labels >= KC).astype(jnp.int32)
    idx2d = (labels - half * KC).reshape(NW * NCH, CHUNK)
    colb = (half << 6)[:, None] + jnp.arange(LANES, dtype=jnp.int32)
    colb = colb.reshape(B * LANES // 128, 128)

    x_p = _pack_rows(inputs.T, KX, KX)
    center_p = _pack_rows(center.T, KC, 7168)

    mesh = plsc.VectorSubcoreMesh(core_axis_name="c", subcore_axis_name="s")
    body = _make_body(BPW, NCH, D)

    partials = pl.kernel(
        body,
        out_type=jax.ShapeDtypeStruct((NW, LANES), jnp.float32),
        mesh=mesh,
        scratch_types=[
            pltpu.VMEM((NCH, CHUNK), jnp.int32),
            pltpu.VMEM((BPW, 64), jnp.float32),
            pltpu.VMEM((2, CHUNK, 128), jnp.float32),
            pltpu.VMEM((BPW * LANES // 128, 128), jnp.int32),
            pltpu.VMEM((LANES,), jnp.float32),
            pltpu.SemaphoreType.DMA,
            pltpu.SemaphoreType.DMA,
        ],
        compiler_params=pltpu.CompilerParams(
            use_tc_tiling_on_sc=False, needs_layout_passes=False),
    )(x_p, idx2d, colb, center_p)

    return jnp.sum(partials) * (1.0 / (B * D))


# speedup vs baseline: 2.3424x; 1.0084x over previous
import jax
import jax.numpy as jnp
from jax import lax
from jax.experimental import pallas as pl
from jax.experimental.pallas import tpu as pltpu
from jax.experimental.pallas import tpu_sc as plsc

NC = 2
NS = 16
NW = NC * NS
LANES = 16
CHUNK = 128


def _pack_rows_block(a_ref, b_ref, out_ref):
    out_ref[...] = jnp.concatenate([a_ref[...].T, b_ref[...].T], axis=1)


def _pack_rows(a_t, k, bkh):
    grid = k // bkh
    return pl.pallas_call(
        _pack_rows_block,
        grid=(grid,),
        in_specs=[
            pl.BlockSpec((64, bkh), lambda g: (0, g)),
            pl.BlockSpec((64, bkh), lambda g: (0, k // bkh + g)),
        ],
        out_specs=pl.BlockSpec((bkh, 128), lambda g: (g, 0)),
        out_shape=jax.ShapeDtypeStruct((k, 128), jnp.float32),
    )(a_t, a_t)


def _make_body(BPW, NCH, D):
    nvec = D // LANES

    def body(x_hbm, idx_hbm, colb_hbm, center_hbm, out_hbm,
             idx_v, x_v, rows_v, colb_v, acc_v, sem0, sem1):
        wid = lax.axis_index("s") * NC + lax.axis_index("c")
        sems = [sem0, sem1]

        pltpu.sync_copy(idx_hbm.at[pl.ds(wid * NCH, NCH)], idx_v)

        def fire(c):
            return pltpu.async_copy(
                center_hbm.at[idx_v.at[c]], rows_v.at[c % 2], sems[c % 2])

        copies = {0: fire(0), 1: fire(1)}
        xrow0 = (wid % NS) * BPW
        xcol0 = (wid // NS) * D
        pltpu.sync_copy(x_hbm.at[pl.ds(xrow0, BPW), pl.ds(xcol0, D)], x_v)
        pltpu.sync_copy(
            colb_hbm.at[pl.ds(wid * (BPW * LANES // 128), BPW * LANES // 128)],
            colb_v)

        zero = jnp.zeros((LANES,), jnp.float32)
        iota16 = lax.iota(jnp.int32, LANES)
        accs = (zero,) * nvec

        for c in range(NCH):
            copies[c].wait()
            buf = c % 2

            def item_body(i, a, _c=c, _buf=buf):
                r = jnp.full((LANES,), i, jnp.int32)
                cb = plsc.load_gather(
                    colb_v,
                    [jnp.full((LANES,), _c * (CHUNK // 8) + (i >> 3), jnp.int32),
                     ((i & 7) << 4) + iota16])
                new = list(a)
                for s in range(nvec):
                    cv = plsc.load_gather(rows_v.at[_buf], [r, cb + s * LANES])
                    xv = x_v[_c * CHUNK + i, pl.ds(s * LANES, LANES)]
                    d = xv - cv
                    new[s] = new[s] + d * d
                return tuple(new)

            accs = lax.fori_loop(0, CHUNK, item_body, accs)
            if c + 2 < NCH:
                copies[c + 2] = fire(c + 2)

        total = accs[0]
        for j in range(1, nvec):
            total = total + accs[j]
        acc_v[...] = total
        pltpu.sync_copy(acc_v, out_hbm.at[wid])

    return body


@jax.jit
def kernel(inputs, labels, center):
    B, D = inputs.shape
    C = center.shape[0]
    BPW = B // NW
    NCH = BPW // CHUNK
    KX = B // 2
    KC = 50176

    labels = labels.astype(jnp.int32)
    half = (labels >= KC).astype(jnp.int32)
    idx2d = (labels - half * KC).reshape(NW * NCH, CHUNK)
    colb = (half << 6)[:, None] + jnp.arange(LANES, dtype=jnp.int32)
    colb = colb.reshape(B * LANES // 128, 128)

    x_p = _pack_rows(inputs.T, KX, 2048)
    center_p = _pack_rows(center.T, KC, 12544)

    mesh = plsc.VectorSubcoreMesh(core_axis_name="c", subcore_axis_name="s")
    body = _make_body(BPW, NCH, D)

    partials = pl.kernel(
        body,
        out_type=jax.ShapeDtypeStruct((NW, LANES), jnp.float32),
        mesh=mesh,
        scratch_types=[
            pltpu.VMEM((NCH, CHUNK), jnp.int32),
            pltpu.VMEM((BPW, 64), jnp.float32),
            pltpu.VMEM((2, CHUNK, 128), jnp.float32),
            pltpu.VMEM((BPW * LANES // 128, 128), jnp.int32),
            pltpu.VMEM((LANES,), jnp.float32),
            pltpu.SemaphoreType.DMA,
            pltpu.SemaphoreType.DMA,
        ],
        compiler_params=pltpu.CompilerParams(
            use_tc_tiling_on_sc=False, needs_layout_passes=False),
    )(x_p, idx2d, colb, center_p)

    return jnp.sum(partials) * (1.0 / (B * D))
